# Initial kernel scaffold; baseline (speedup 1.0000x reference)
import jax
import jax.numpy as jnp
from jax.experimental import pallas as pl


def kernel(input_ids, table, learned):
    raise NotImplementedError("write your pallas kernel here")


# speedup vs baseline: 4.2297x; 4.2297x over previous
import functools

import jax
import jax.numpy as jnp
from jax import lax
from jax.experimental import pallas as pl
from jax.experimental.pallas import tpu as pltpu
from jax.experimental.pallas import tpu_sc as plsc

B = 4096
S = 50
D = 64
P = 20
T = P + S

NC = 2
NS = 16
NW = NC * NS
NB = B // NW
G = 8


def _soft_embed(ids_hbm, table_hbm, learned_hbm, out_hbm, idx_v, obuf, sem):
    wid = lax.axis_index("s") * NC + lax.axis_index("c")
    b0 = wid * NB
    pltpu.sync_copy(ids_hbm.at[pl.ds(b0, NB)], idx_v)
    for g in range(G):
        pltpu.sync_copy(learned_hbm, obuf.at[g, pl.ds(0, P)])

    def body(og, carry):
        copies = []
        for g in range(G):
            copies.append(
                pltpu.async_copy(
                    table_hbm.at[idx_v.at[og * G + g]],
                    obuf.at[g, pl.ds(P, S)],
                    sem,
                )
            )
        for c in copies:
            c.wait()
        pltpu.sync_copy(obuf, out_hbm.at[pl.ds(b0 + og * G, G)])
        return carry

    lax.fori_loop(0, NB // G, body, 0)


def kernel(input_ids, table, learned):
    mesh = plsc.VectorSubcoreMesh(core_axis_name="c", subcore_axis_name="s")
    run = functools.partial(
        pl.kernel,
        mesh=mesh,
        out_type=jax.ShapeDtypeStruct((B, T, D), jnp.float32),
        scratch_types=[
            pltpu.VMEM((NB, S), jnp.int32),
            pltpu.VMEM((G, T, D), jnp.float32),
            pltpu.SemaphoreType.DMA,
        ],
        compiler_params=pltpu.CompilerParams(use_tc_tiling_on_sc=False),
    )(_soft_embed)
    return run(input_ids, table, learned)
